# Initial kernel scaffold; baseline (speedup 1.0000x reference)
import jax
import jax.numpy as jnp
from jax.experimental import pallas as pl


def kernel(x, router_logits):
    raise NotImplementedError("write your pallas kernel here")


# speedup vs baseline: 13.9330x; 13.9330x over previous
import jax
import jax.numpy as jnp
from jax.experimental import pallas as pl
from jax.experimental.pallas import tpu as pltpu

_NUM_EXPERTS = 16
_TOKENS_PER_BLOCK = 512


def _dispatch_combine_block(logits_ref, x_ref, out_ref):
    logits = logits_ref[...]
    m = jnp.max(logits, axis=-1, keepdims=True)
    e = jnp.exp(logits - m)
    probs = e / jnp.sum(e, axis=-1, keepdims=True)
    v1 = jnp.max(probs, axis=-1)
    idx1 = jnp.argmax(probs, axis=-1)
    cols = jax.lax.broadcasted_iota(jnp.int32, probs.shape, 1)
    v2 = jnp.max(jnp.where(cols == idx1[:, None], -jnp.inf, probs), axis=-1)
    denom = v1 + v2
    gate_sum = v1 / denom + v2 / denom
    out_ref[...] = x_ref[...] * gate_sum[:, None]


def kernel(x, router_logits):
    tokens, d_model = x.shape
    bt = _TOKENS_PER_BLOCK
    grid = (tokens // bt,)
    return pl.pallas_call(
        _dispatch_combine_block,
        grid=grid,
        in_specs=[
            pl.BlockSpec((bt, _NUM_EXPERTS), lambda i: (i, 0)),
            pl.BlockSpec((bt, d_model), lambda i: (i, 0)),
        ],
        out_specs=pl.BlockSpec((bt, d_model), lambda i: (i, 0)),
        out_shape=jax.ShapeDtypeStruct((tokens, d_model), x.dtype),
        compiler_params=pltpu.CompilerParams(
            dimension_semantics=("arbitrary",),
        ),
    )(router_logits, x)


# speedup vs baseline: 14.1435x; 1.0151x over previous
import jax
import jax.numpy as jnp
from jax.experimental import pallas as pl
from jax.experimental.pallas import tpu as pltpu

_NUM_EXPERTS = 16
_TOKENS_PER_BLOCK = 1024


def _dispatch_combine_block(logits_ref, x_ref, out_ref):
    logits = logits_ref[...]
    m = jnp.max(logits, axis=-1, keepdims=True)
    e = jnp.exp(logits - m)
    probs = e / jnp.sum(e, axis=-1, keepdims=True)
    v1 = jnp.max(probs, axis=-1)
    idx1 = jnp.argmax(probs, axis=-1)
    cols = jax.lax.broadcasted_iota(jnp.int32, probs.shape, 1)
    v2 = jnp.max(jnp.where(cols == idx1[:, None], -jnp.inf, probs), axis=-1)
    denom = v1 + v2
    gate_sum = v1 / denom + v2 / denom
    out_ref[...] = x_ref[...] * gate_sum[:, None]


def kernel(x, router_logits):
    tokens, d_model = x.shape
    bt = _TOKENS_PER_BLOCK
    grid = (tokens // bt,)
    return pl.pallas_call(
        _dispatch_combine_block,
        grid=grid,
        in_specs=[
            pl.BlockSpec((bt, _NUM_EXPERTS), lambda i: (i, 0)),
            pl.BlockSpec((bt, d_model), lambda i: (i, 0)),
        ],
        out_specs=pl.BlockSpec((bt, d_model), lambda i: (i, 0)),
        out_shape=jax.ShapeDtypeStruct((tokens, d_model), x.dtype),
        compiler_params=pltpu.CompilerParams(
            dimension_semantics=("arbitrary",),
        ),
    )(router_logits, x)


# speedup vs baseline: 14.2095x; 1.0047x over previous
import jax
import jax.numpy as jnp
from jax.experimental import pallas as pl
from jax.experimental.pallas import tpu as pltpu

_NUM_EXPERTS = 16
_TOKENS_PER_BLOCK = 1024


def _dispatch_combine_block(logits_ref, x_ref, out_ref):
    logits = logits_ref[...]
    m = jnp.max(logits, axis=-1, keepdims=True)
    e = jnp.exp(logits - m)
    probs = e / jnp.sum(e, axis=-1, keepdims=True)
    v1 = jnp.max(probs, axis=-1)
    idx1 = jnp.argmax(probs, axis=-1)
    cols = jax.lax.broadcasted_iota(jnp.int32, probs.shape, 1)
    v2 = jnp.max(jnp.where(cols == idx1[:, None], -jnp.inf, probs), axis=-1)
    denom = v1 + v2
    gate_sum = v1 / denom + v2 / denom
    out_ref[...] = x_ref[...] * gate_sum[:, None]


def kernel(x, router_logits):
    tokens, d_model = x.shape
    bt = _TOKENS_PER_BLOCK
    grid = (tokens // bt,)
    return pl.pallas_call(
        _dispatch_combine_block,
        grid=grid,
        in_specs=[
            pl.BlockSpec((bt, _NUM_EXPERTS), lambda i: (i, 0)),
            pl.BlockSpec((bt, d_model), lambda i: (i, 0)),
        ],
        out_specs=pl.BlockSpec((bt, d_model), lambda i: (i, 0)),
        out_shape=jax.ShapeDtypeStruct((tokens, d_model), x.dtype),
        compiler_params=pltpu.CompilerParams(
            dimension_semantics=("parallel",),
        ),
    )(router_logits, x)
